# baseline (speedup 1.0000x reference)
import jax
import jax.numpy as jnp
from jax import lax
from jax.experimental import pallas as pl
from jax.experimental.pallas import tpu as pltpu
from jax.experimental.pallas import tpu_sc as plsc

_B = 16384
_F = 26
_NW = 32
_BW = _B // _NW
_K = _F * _BW
_C = 128
_R = _K // _C


def _body(idx_hbm, table_hbm, out_hbm, idx_v, vals_v, out_v, sem):
    wid = lax.axis_index("s") * 2 + lax.axis_index("c")

    pltpu.sync_copy(idx_hbm.at[wid], idx_v)

    w = 16

    @pl.loop(0, _R)
    def _fire(j):
        pltpu.async_copy(table_hbm.at[idx_v.at[j]], vals_v.at[j], sem)

        @pl.when(j >= w)
        def _():
            pltpu.make_async_copy(
                table_hbm.at[idx_v.at[0]], vals_v.at[0], sem
            ).wait()

    @pl.loop(0, w)
    def _drain(j):
        pltpu.make_async_copy(
            table_hbm.at[idx_v.at[0]], vals_v.at[0], sem
        ).wait()

    lanes26 = lax.iota(jnp.int32, 16) * _F
    for c in range(_BW // 16):
        base = lanes26 + (c * 16 * _F)
        acc = plsc.load_gather(
            vals_v,
            [lax.shift_right_logical(base, 7), lax.bitwise_and(base, 127)],
        )
        for f in range(1, _F):
            p = base + f
            acc = acc + plsc.load_gather(
                vals_v,
                [lax.shift_right_logical(p, 7), lax.bitwise_and(p, 127)],
            )
        out_v[pl.ds(16 * c, 16)] = acc

    pltpu.sync_copy(out_v, out_hbm.at[pl.ds(wid * _BW, _BW)])


@jax.jit
def _fl_kernel(idx_all, table):
    mesh = plsc.VectorSubcoreMesh(core_axis_name="c", subcore_axis_name="s")
    k = pl.kernel(
        _body,
        out_type=jax.ShapeDtypeStruct((_B,), jnp.float32),
        mesh=mesh,
        compiler_params=pltpu.CompilerParams(needs_layout_passes=False),
        scratch_types=[
            pltpu.VMEM((_R, _C), jnp.int32),
            pltpu.VMEM((_R, _C), jnp.float32),
            pltpu.VMEM((_BW,), jnp.float32),
            pltpu.SemaphoreType.DMA,
        ],
    )
    return k(idx_all, table)


def kernel(x, fc_weight, bias):
    idx_all = x.astype(jnp.int32).reshape(_NW, _R, _C)
    table = fc_weight.reshape(-1)
    out = _fl_kernel(idx_all, table)
    return out.reshape(_B, 1) + bias[None, :]


# speedup vs baseline: 1.1045x; 1.1045x over previous
import jax
import jax.numpy as jnp
from jax import lax
from jax.experimental import pallas as pl
from jax.experimental.pallas import tpu as pltpu
from jax.experimental.pallas import tpu_sc as plsc

_B = 16384
_F = 26
_NW = 32
_BW = _B // _NW
_K = _F * _BW
_C = 128
_R = _K // _C
_L = 16


def _body(idx_hbm, table_hbm, bias_hbm, out_hbm, idx_v, vals_v, out_v,
          bias_v, sem):
    wid = lax.axis_index("s") * 2 + lax.axis_index("c")

    pltpu.sync_copy(idx_hbm.at[wid], idx_v)
    pltpu.sync_copy(bias_hbm, bias_v)

    w = 16

    @pl.loop(0, _R)
    def _fire(j):
        pltpu.async_copy(table_hbm.at[idx_v.at[j]], vals_v.at[j], sem)

        @pl.when(j >= w)
        def _():
            pltpu.make_async_copy(
                table_hbm.at[idx_v.at[0]], vals_v.at[0], sem
            ).wait()

    @pl.loop(0, w)
    def _drain(j):
        pltpu.make_async_copy(
            table_hbm.at[idx_v.at[0]], vals_v.at[0], sem
        ).wait()

    bias_vec = bias_v[pl.ds(0, _L)]
    for c in range(_BW // _L):
        r, l = c // 8, _L * (c % 8)
        acc = vals_v[0 + r, pl.ds(l, _L)] + bias_vec
        for f in range(1, _F):
            acc = acc + vals_v[f * 4 + r, pl.ds(l, _L)]
        out_v[pl.ds(_L * c, _L)] = acc

    pltpu.sync_copy(out_v, out_hbm.at[pl.ds(wid * _BW, _BW)])


@jax.jit
def _fl_kernel(idx_all, table, bias16):
    mesh = plsc.VectorSubcoreMesh(core_axis_name="c", subcore_axis_name="s")
    k = pl.kernel(
        _body,
        out_type=jax.ShapeDtypeStruct((_B,), jnp.float32),
        mesh=mesh,
        scratch_types=[
            pltpu.VMEM((_R, _C), jnp.int32),
            pltpu.VMEM((_R, _C), jnp.float32),
            pltpu.VMEM((_BW,), jnp.float32),
            pltpu.VMEM((_L,), jnp.float32),
            pltpu.SemaphoreType.DMA,
        ],
    )
    return k(idx_all, table, bias16)


def kernel(x, fc_weight, bias):
    idx_all = (
        x.astype(jnp.int32)
        .reshape(_NW, _BW, _F)
        .transpose(0, 2, 1)
        .reshape(_NW, _R, _C)
    )
    table = fc_weight.reshape(-1)
    bias16 = jnp.broadcast_to(bias, (_L,))
    out = _fl_kernel(idx_all, table, bias16)
    return out.reshape(_B, 1)


# speedup vs baseline: 2.0914x; 1.8935x over previous
import jax
import jax.numpy as jnp
from jax import lax
from jax.experimental import pallas as pl
from jax.experimental.pallas import tpu as pltpu
from jax.experimental.pallas import tpu_sc as plsc

_B = 16384
_F = 26
_NW = 32
_BW = _B // _NW
_K = _F * _BW
_C = 128
_R = _K // _C
_L = 16


def _body(idx_hbm, table_hbm, bias_hbm, out_hbm, idx_v, vals_v, out_v,
          bias_v, sem):
    wid = lax.axis_index("s") * 2 + lax.axis_index("c")

    pltpu.sync_copy(idx_hbm.at[wid], idx_v)
    pltpu.sync_copy(bias_hbm, bias_v)

    w = 16

    @pl.loop(0, _R)
    def _fire(j):
        pltpu.async_copy(table_hbm.at[idx_v.at[j]], vals_v.at[j], sem)

        @pl.when(j >= w)
        def _():
            pltpu.make_async_copy(
                table_hbm.at[idx_v.at[0]], vals_v.at[0], sem
            ).wait()

    @pl.loop(0, w)
    def _drain(j):
        pltpu.make_async_copy(
            table_hbm.at[idx_v.at[0]], vals_v.at[0], sem
        ).wait()

    bias_vec = bias_v[pl.ds(0, _L)]
    for c in range(_BW // _L):
        r, l = c // 8, _L * (c % 8)
        acc = vals_v[0 + r, 0, pl.ds(l, _L)] + bias_vec
        for f in range(1, _F):
            acc = acc + vals_v[f * 4 + r, 0, pl.ds(l, _L)]
        out_v[pl.ds(_L * c, _L)] = acc

    pltpu.sync_copy(out_v, out_hbm.at[pl.ds(wid * _BW, _BW)])


@jax.jit
def _fl_kernel(idx_all, table, bias16):
    mesh = plsc.VectorSubcoreMesh(core_axis_name="c", subcore_axis_name="s")
    k = pl.kernel(
        _body,
        out_type=jax.ShapeDtypeStruct((_B,), jnp.float32),
        mesh=mesh,
        scratch_types=[
            pltpu.VMEM((_R, 1, _C), jnp.int32),
            pltpu.VMEM((_R, 1, _C), jnp.float32),
            pltpu.VMEM((_BW,), jnp.float32),
            pltpu.VMEM((_L,), jnp.float32),
            pltpu.SemaphoreType.DMA,
        ],
    )
    return k(idx_all, table, bias16)


def kernel(x, fc_weight, bias):
    idx_all = (
        x.astype(jnp.int32)
        .reshape(_NW, _BW, _F)
        .transpose(0, 2, 1)
        .reshape(_NW, _R, 1, _C)
    )
    bias16 = jnp.broadcast_to(bias, (_L,))
    out = _fl_kernel(idx_all, fc_weight.T, bias16)
    return out.reshape(_B, 1)


# speedup vs baseline: 2.3499x; 1.1236x over previous
import jax
import jax.numpy as jnp
from jax import lax
from jax.experimental import pallas as pl
from jax.experimental.pallas import tpu as pltpu
from jax.experimental.pallas import tpu_sc as plsc

_B = 16384
_F = 26
_NW = 32
_BW = _B // _NW
_K = _F * _BW
_C = 128
_R = _K // _C
_L = 16


def _body(idx_hbm, table_hbm, bias_hbm, out_hbm, idx_v, vals_v, out_v,
          bias_v, sem):
    wid = lax.axis_index("s") * 2 + lax.axis_index("c")

    pltpu.sync_copy(idx_hbm.at[wid], idx_v)
    pltpu.sync_copy(bias_hbm, bias_v)

    @pl.loop(0, _R)
    def _fire(j):
        pltpu.async_copy(table_hbm.at[idx_v.at[j]], vals_v.at[j], sem)

    @pl.loop(0, _R)
    def _drain(j):
        pltpu.make_async_copy(
            table_hbm.at[idx_v.at[0]], vals_v.at[0], sem
        ).wait()

    bias_vec = bias_v[pl.ds(0, _L)]
    for c in range(_BW // _L):
        r, l = c // 8, _L * (c % 8)
        acc = vals_v[0 + r, 0, pl.ds(l, _L)] + bias_vec
        for f in range(1, _F):
            acc = acc + vals_v[f * 4 + r, 0, pl.ds(l, _L)]
        out_v[pl.ds(_L * c, _L)] = acc

    pltpu.sync_copy(out_v, out_hbm.at[pl.ds(wid * _BW, _BW)])


@jax.jit
def _fl_kernel(idx_all, table, bias16):
    mesh = plsc.VectorSubcoreMesh(core_axis_name="c", subcore_axis_name="s")
    k = pl.kernel(
        _body,
        out_type=jax.ShapeDtypeStruct((_B,), jnp.float32),
        mesh=mesh,
        scratch_types=[
            pltpu.VMEM((_R, 1, _C), jnp.int32),
            pltpu.VMEM((_R, 1, _C), jnp.float32),
            pltpu.VMEM((_BW,), jnp.float32),
            pltpu.VMEM((_L,), jnp.float32),
            pltpu.SemaphoreType.DMA,
        ],
    )
    return k(idx_all, table, bias16)


def kernel(x, fc_weight, bias):
    idx_all = (
        x.astype(jnp.int32)
        .reshape(_NW, _BW, _F)
        .transpose(0, 2, 1)
        .reshape(_NW, _R, 1, _C)
    )
    bias16 = jnp.broadcast_to(bias, (_L,))
    out = _fl_kernel(idx_all, fc_weight.T, bias16)
    return out.reshape(_B, 1)


# speedup vs baseline: 2.3606x; 1.0046x over previous
import jax
import jax.numpy as jnp
from jax import lax
from jax.experimental import pallas as pl
from jax.experimental.pallas import tpu as pltpu
from jax.experimental.pallas import tpu_sc as plsc

_B = 16384
_F = 26
_NW = 32
_BW = _B // _NW
_K = _F * _BW
_C = 128
_R = _K // _C
_L = 16


def _body(idx_hbm, table_hbm, bias_hbm, out_hbm, idx_v, vals_v, out_v,
          bias_v, sem):
    wid = lax.axis_index("s") * 2 + lax.axis_index("c")

    pltpu.sync_copy(idx_hbm.at[wid], idx_v)
    pltpu.sync_copy(bias_hbm, bias_v)

    @pl.loop(0, _R // 8)
    def _fire(g):
        base = g * 8
        for u in range(8):
            pltpu.async_copy(
                table_hbm.at[idx_v.at[base + u]], vals_v.at[base + u], sem
            )

    pltpu.make_async_copy(idx_hbm.at[wid], vals_v, sem).wait()

    bias_vec = bias_v[pl.ds(0, _L)]
    for c in range(_BW // _L):
        r, l = c // 8, _L * (c % 8)
        acc = vals_v[0 + r, 0, pl.ds(l, _L)] + bias_vec
        for f in range(1, _F):
            acc = acc + vals_v[f * 4 + r, 0, pl.ds(l, _L)]
        out_v[pl.ds(_L * c, _L)] = acc

    pltpu.sync_copy(out_v, out_hbm.at[pl.ds(wid * _BW, _BW)])


@jax.jit
def _fl_kernel(idx_all, table, bias16):
    mesh = plsc.VectorSubcoreMesh(core_axis_name="c", subcore_axis_name="s")
    k = pl.kernel(
        _body,
        out_type=jax.ShapeDtypeStruct((_B,), jnp.float32),
        mesh=mesh,
        scratch_types=[
            pltpu.VMEM((_R, 1, _C), jnp.int32),
            pltpu.VMEM((_R, 1, _C), jnp.float32),
            pltpu.VMEM((_BW,), jnp.float32),
            pltpu.VMEM((_L,), jnp.float32),
            pltpu.SemaphoreType.DMA,
        ],
    )
    return k(idx_all, table, bias16)


def kernel(x, fc_weight, bias):
    idx_all = (
        x.astype(jnp.int32)
        .reshape(_NW, _BW, _F)
        .transpose(0, 2, 1)
        .reshape(_NW, _R, 1, _C)
    )
    bias16 = jnp.broadcast_to(bias, (_L,))
    out = _fl_kernel(idx_all, fc_weight.T, bias16)
    return out.reshape(_B, 1)


# speedup vs baseline: 2.8356x; 1.2012x over previous
import jax
import jax.numpy as jnp
from jax import lax
from jax.experimental import pallas as pl
from jax.experimental.pallas import tpu as pltpu
from jax.experimental.pallas import tpu_sc as plsc

_B = 16384
_F = 26
_NW = 32
_BW = _B // _NW
_K = _F * _BW
_C = 128
_R = _K // _C
_L = 16


_TCH = 62592
_TLAST = 1000000 - 15 * _TCH


def _body(idx_hbm, table_hbm, bias_hbm, out_hbm, idx_v, vals_v, out_v,
          bias_v, table_sh, sem):
    sid = lax.axis_index("s")
    wid = sid * 2 + lax.axis_index("c")

    pltpu.sync_copy(idx_hbm.at[wid], idx_v)
    pltpu.sync_copy(bias_hbm, bias_v)

    @pl.when(sid < 15)
    def _():
        start = sid * _TCH
        pltpu.sync_copy(
            table_hbm.at[0, pl.ds(start, _TCH)],
            table_sh.at[0, pl.ds(start, _TCH)],
        )

    @pl.when(sid == 15)
    def _():
        pltpu.sync_copy(
            table_hbm.at[0, pl.ds(15 * _TCH, _TLAST)],
            table_sh.at[0, pl.ds(15 * _TCH, _TLAST)],
        )

    plsc.subcore_barrier()

    @pl.loop(0, _R // 8)
    def _fire(g):
        base = g * 8
        for u in range(8):
            pltpu.async_copy(
                table_sh.at[idx_v.at[base + u]], vals_v.at[base + u], sem
            )

    pltpu.make_async_copy(idx_hbm.at[wid], vals_v, sem).wait()

    bias_vec = bias_v[pl.ds(0, _L)]
    for c in range(_BW // _L):
        r, l = c // 8, _L * (c % 8)
        acc = vals_v[0 + r, 0, pl.ds(l, _L)] + bias_vec
        for f in range(1, _F):
            acc = acc + vals_v[f * 4 + r, 0, pl.ds(l, _L)]
        out_v[pl.ds(_L * c, _L)] = acc

    pltpu.sync_copy(out_v, out_hbm.at[pl.ds(wid * _BW, _BW)])


@jax.jit
def _fl_kernel(idx_all, table, bias16):
    mesh = plsc.VectorSubcoreMesh(core_axis_name="c", subcore_axis_name="s")
    k = pl.kernel(
        _body,
        out_type=jax.ShapeDtypeStruct((_B,), jnp.float32),
        mesh=mesh,
        scratch_types=[
            pltpu.VMEM((_R, 1, _C), jnp.int32),
            pltpu.VMEM((_R, 1, _C), jnp.float32),
            pltpu.VMEM((_BW,), jnp.float32),
            pltpu.VMEM((_L,), jnp.float32),
            pltpu.VMEM_SHARED((1, 1000000), jnp.float32),
            pltpu.SemaphoreType.DMA,
        ],
    )
    return k(idx_all, table, bias16)


def kernel(x, fc_weight, bias):
    idx_all = (
        x.astype(jnp.int32)
        .reshape(_NW, _BW, _F)
        .transpose(0, 2, 1)
        .reshape(_NW, _R, 1, _C)
    )
    bias16 = jnp.broadcast_to(bias, (_L,))
    out = _fl_kernel(idx_all, fc_weight.T, bias16)
    return out.reshape(_B, 1)


# speedup vs baseline: 3.1185x; 1.0998x over previous
import jax
import jax.numpy as jnp
from jax import lax
from jax.experimental import pallas as pl
from jax.experimental.pallas import tpu as pltpu
from jax.experimental.pallas import tpu_sc as plsc

_B = 16384
_F = 26
_NW = 32
_BW = _B // _NW
_K = _F * _BW
_C = 128
_R = _K // _C
_L = 16


_TCH = 62592
_TLAST = 1000000 - 15 * _TCH


def _body(idx_hbm, table_hbm, bias_hbm, out_hbm, idx_v, acc_v, out_v,
          bias_v, table_sh, sem, sem2):
    sid = lax.axis_index("s")
    wid = sid * 2 + lax.axis_index("c")

    pltpu.async_copy(idx_hbm.at[wid], idx_v, sem2)
    pltpu.async_copy(bias_hbm, bias_v, sem2)

    zero = jnp.zeros((_L,), jnp.float32)
    for q in range(4):
        for l in range(0, _C, _L):
            acc_v[q, 0, pl.ds(l, _L)] = zero

    @pl.when(sid < 15)
    def _():
        start = sid * _TCH
        pltpu.sync_copy(
            table_hbm.at[0, pl.ds(start, _TCH)],
            table_sh.at[0, pl.ds(start, _TCH)],
        )

    @pl.when(sid == 15)
    def _():
        pltpu.sync_copy(
            table_hbm.at[0, pl.ds(15 * _TCH, _TLAST)],
            table_sh.at[0, pl.ds(15 * _TCH, _TLAST)],
        )

    plsc.subcore_barrier()

    pltpu.make_async_copy(idx_hbm.at[wid], idx_v, sem2).wait()
    pltpu.make_async_copy(bias_hbm, bias_v, sem2).wait()

    @pl.loop(0, _R // 8)
    def _fire(g):
        base = g * 8
        for u in range(8):
            pltpu.async_copy(
                table_sh.at[idx_v.at[base + u]], acc_v.at[u % 4], sem,
                add=True,
            )

    pltpu.make_async_copy(idx_hbm.at[wid], idx_v, sem).wait()

    bias_vec = bias_v[pl.ds(0, _L)]
    for c in range(_BW // _L):
        q, l = c // 8, _L * (c % 8)
        out_v[pl.ds(_L * c, _L)] = acc_v[q, 0, pl.ds(l, _L)] + bias_vec

    pltpu.sync_copy(out_v, out_hbm.at[pl.ds(wid * _BW, _BW)])


@jax.jit
def _fl_kernel(idx_all, table, bias16):
    mesh = plsc.VectorSubcoreMesh(core_axis_name="c", subcore_axis_name="s")
    k = pl.kernel(
        _body,
        out_type=jax.ShapeDtypeStruct((_B,), jnp.float32),
        mesh=mesh,
        scratch_types=[
            pltpu.VMEM((_R, 1, _C), jnp.int32),
            pltpu.VMEM((4, 1, _C), jnp.float32),
            pltpu.VMEM((_BW,), jnp.float32),
            pltpu.VMEM((_L,), jnp.float32),
            pltpu.VMEM_SHARED((1, 1000000), jnp.float32),
            pltpu.SemaphoreType.DMA,
            pltpu.SemaphoreType.DMA,
        ],
    )
    return k(idx_all, table, bias16)


def kernel(x, fc_weight, bias):
    idx_all = (
        x.astype(jnp.int32)
        .reshape(_NW, _BW, _F)
        .transpose(0, 2, 1)
        .reshape(_NW, _R, 1, _C)
    )
    bias16 = jnp.broadcast_to(bias, (_L,))
    out = _fl_kernel(idx_all, fc_weight.T, bias16)
    return out.reshape(_B, 1)


# speedup vs baseline: 3.1268x; 1.0026x over previous
import jax
import jax.numpy as jnp
from jax import lax
from jax.experimental import pallas as pl
from jax.experimental.pallas import tpu as pltpu
from jax.experimental.pallas import tpu_sc as plsc

_B = 16384
_F = 26
_NW = 32
_BW = _B // _NW
_K = _F * _BW
_C = 128
_R = _K // _C
_L = 16


_TCH = 62592
_TLAST = 1000000 - 15 * _TCH


def _body(idx_hbm, table_hbm, bias_hbm, out_hbm, idx_v, acc_v, out_v,
          bias_v, table_sh, sem, sem2):
    sid = lax.axis_index("s")
    wid = sid * 2 + lax.axis_index("c")

    pltpu.async_copy(idx_hbm.at[wid], idx_v, sem2)
    pltpu.async_copy(bias_hbm, bias_v, sem2)

    zero = jnp.zeros((_L,), jnp.float32)
    for q in range(4):
        for l in range(0, _C, _L):
            acc_v[q, 0, pl.ds(l, _L)] = zero

    @pl.when(sid < 15)
    def _():
        start = sid * _TCH
        pltpu.sync_copy(
            table_hbm.at[0, pl.ds(start, _TCH)],
            table_sh.at[0, pl.ds(start, _TCH)],
        )

    @pl.when(sid == 15)
    def _():
        pltpu.sync_copy(
            table_hbm.at[0, pl.ds(15 * _TCH, _TLAST)],
            table_sh.at[0, pl.ds(15 * _TCH, _TLAST)],
        )

    plsc.subcore_barrier()

    pltpu.make_async_copy(idx_hbm.at[wid], idx_v, sem2).wait()
    pltpu.make_async_copy(bias_hbm, bias_v, sem2).wait()

    @pl.loop(0, _R // 8)
    def _fire(g):
        base = g * 8
        for u in range(8):
            pltpu.async_copy(
                table_sh.at[idx_v.at[base + u]], acc_v.at[u % 4], sem,
                add=True,
            )

    pltpu.make_async_copy(idx_hbm.at[wid], idx_v, sem).wait()

    bias_vec = bias_v[pl.ds(0, _L)]
    for c in range(_BW // _L):
        q, l = c // 8, _L * (c % 8)
        out_v[pl.ds(_L * c, _L)] = acc_v[q, 0, pl.ds(l, _L)] + bias_vec

    pltpu.sync_copy(out_v, out_hbm.at[0, pl.ds(wid * _BW, _BW)])


@jax.jit
def _fl_kernel(idx_all, table, bias16):
    mesh = plsc.VectorSubcoreMesh(core_axis_name="c", subcore_axis_name="s")
    k = pl.kernel(
        _body,
        out_type=jax.ShapeDtypeStruct((1, _B), jnp.float32),
        mesh=mesh,
        scratch_types=[
            pltpu.VMEM((_R, 1, _C), jnp.int32),
            pltpu.VMEM((4, 1, _C), jnp.float32),
            pltpu.VMEM((_BW,), jnp.float32),
            pltpu.VMEM((_L,), jnp.float32),
            pltpu.VMEM_SHARED((1, 1000000), jnp.float32),
            pltpu.SemaphoreType.DMA,
            pltpu.SemaphoreType.DMA,
        ],
    )
    return k(idx_all, table, bias16)


def kernel(x, fc_weight, bias):
    idx_all = (
        x.astype(jnp.int32)
        .reshape(_NW, _BW, _F)
        .transpose(0, 2, 1)
        .reshape(_NW, _R, 1, _C)
    )
    bias16 = jnp.broadcast_to(bias, (_L,))
    out = _fl_kernel(idx_all, fc_weight.T, bias16)
    return out.reshape(_B, 1)
